# Initial kernel scaffold; baseline (speedup 1.0000x reference)
import jax
import jax.numpy as jnp
from jax.experimental import pallas as pl


def kernel(w1, b1, w2, b2, fc1_w, fc1_b, fc2_w, fc2_b, mask_l, mask_r, x):
    raise NotImplementedError("write your pallas kernel here")


# speedup vs baseline: 1.0156x; 1.0156x over previous
import jax
import jax.numpy as jnp
from jax.experimental import pallas as pl
from jax.experimental.pallas import tpu as pltpu

H = W = 28
NPIX = H * W
MARGIN = 32
S = NPIX + MARGIN
CPAD = 128
C2 = 64
HO = WO = 7
NPOOL = HO * WO
FC_IN = C2 * NPOOL
PROWS = 56
PCOLS = 704
TB = 8
TALL = TB * S
XR = TALL + 2 * MARGIN
TBF = 256
TAPS = [(di, dj) for di in range(3) for dj in range(3)]


def _conv_pool_k(x_ref, tml_ref, tmr_ref, w1_ref, b1_ref, w2p_ref, w2l_ref,
                 b2_ref, p_ref, o_ref, xm_ref, hm_ref, acc_ref):
    for i in range(TB):
        xm_ref[pl.ds(MARGIN + i * S, NPIX), :] = x_ref[i]
    xm_ref[pl.ds(0, MARGIN), :] = jnp.zeros((MARGIN, 1), jnp.float32)
    for i in range(TB):
        gap = MARGIN + i * S + NPIX
        glen = MARGIN if i < TB - 1 else 2 * MARGIN
        xm_ref[pl.ds(gap, glen), :] = jnp.zeros((glen, 1), jnp.float32)

    acc = jnp.zeros((TALL, CPAD), jnp.float32)
    for t, (di, dj) in enumerate(TAPS):
        off = (di - 1) * W + (dj - 1)
        tap = xm_ref[pl.ds(MARGIN + off, TALL), :]
        if dj == 0:
            tap = tap * tml_ref[...]
        elif dj == 2:
            tap = tap * tmr_ref[...]
        acc = acc + tap * w1_ref[pl.ds(t, 1), :]
    h1 = jnp.maximum(acc + b1_ref[...], 0.0).astype(jnp.bfloat16)
    hm_ref[pl.ds(MARGIN, TALL), :] = h1

    def _zero_gaps():
        zb = jnp.zeros((2 * MARGIN, CPAD), jnp.bfloat16)
        hm_ref[pl.ds(0, MARGIN), :] = zb[:MARGIN]
        for i in range(TB):
            gap = MARGIN + i * S + NPIX
            glen = MARGIN if i < TB - 1 else 2 * MARGIN
            hm_ref[pl.ds(gap, glen), :] = zb[:glen]

    _zero_gaps()

    def tap2(t):
        di, dj = TAPS[t]
        off = (di - 1) * W + (dj - 1)
        v = hm_ref[pl.ds(MARGIN + off, TALL), :]
        if dj == 0:
            v = v * tml_ref[...]
        elif dj == 2:
            v = v * tmr_ref[...]
        return v

    acc2 = jnp.zeros((TALL, CPAD), jnp.float32)
    for g in range(4):
        op = jnp.concatenate([tap2(2 * g), tap2(2 * g + 1)], axis=1)
        acc2 = acc2 + jnp.dot(op, w2p_ref[g], preferred_element_type=jnp.float32)
    acc2 = acc2 + jnp.dot(tap2(8), w2l_ref[...],
                          preferred_element_type=jnp.float32)
    h2 = jnp.maximum(acc2 + b2_ref[...], 0.0).astype(jnp.bfloat16)
    hm_ref[pl.ds(MARGIN, TALL), :] = h2
    _zero_gaps()

    hmax = jnp.maximum(
        jnp.maximum(hm_ref[pl.ds(MARGIN + 0, TALL), :],
                    hm_ref[pl.ds(MARGIN + 1, TALL), :]),
        jnp.maximum(hm_ref[pl.ds(MARGIN + 2, TALL), :],
                    hm_ref[pl.ds(MARGIN + 3, TALL), :]))
    acc_ref[...] = hmax.astype(jnp.float32)
    vl2 = TALL - 3 * W
    vm = jnp.maximum(
        jnp.maximum(acc_ref[pl.ds(0 * W, vl2), :],
                    acc_ref[pl.ds(1 * W, vl2), :]),
        jnp.maximum(acc_ref[pl.ds(2 * W, vl2), :],
                    acc_ref[pl.ds(3 * W, vl2), :]))
    hm_ref[pl.ds(0, vl2), :] = vm.astype(jnp.bfloat16)

    for i in range(TB):
        pooled = jnp.dot(p_ref[...], hm_ref[pl.ds(i * S, PCOLS), :],
                         preferred_element_type=jnp.float32)
        o_ref[i] = pooled[:NPOOL, :C2].astype(jnp.bfloat16)


def _conv_pool(xp, tml, tmr, w1, b1, w2p, w2l, b2, pmat):
    bp = xp.shape[0]
    return pl.pallas_call(
        _conv_pool_k,
        out_shape=jax.ShapeDtypeStruct((bp, NPOOL, C2), jnp.bfloat16),
        grid_spec=pltpu.PrefetchScalarGridSpec(
            num_scalar_prefetch=0,
            grid=(bp // TB,),
            in_specs=[
                pl.BlockSpec((TB, NPIX, 1), lambda b: (b, 0, 0)),
                pl.BlockSpec((TALL, 1), lambda b: (0, 0)),
                pl.BlockSpec((TALL, 1), lambda b: (0, 0)),
                pl.BlockSpec((9, CPAD), lambda b: (0, 0)),
                pl.BlockSpec((1, CPAD), lambda b: (0, 0)),
                pl.BlockSpec((4, 256, CPAD), lambda b: (0, 0, 0)),
                pl.BlockSpec((CPAD, CPAD), lambda b: (0, 0)),
                pl.BlockSpec((1, CPAD), lambda b: (0, 0)),
                pl.BlockSpec((PROWS, PCOLS), lambda b: (0, 0)),
            ],
            out_specs=pl.BlockSpec((TB, NPOOL, C2), lambda b: (b, 0, 0)),
            scratch_shapes=[
                pltpu.VMEM((XR, 1), jnp.float32),
                pltpu.VMEM((XR, CPAD), jnp.bfloat16),
                pltpu.VMEM((TALL, CPAD), jnp.float32),
            ]),
        compiler_params=pltpu.CompilerParams(
            dimension_semantics=("parallel",)),
    )(xp, tml, tmr, w1, b1, w2p, w2l, b2, pmat)


def _fc_k(x_ref, w1_ref, b1_ref, w2_ref, b2_ref, o_ref):
    h = jnp.dot(x_ref[...], w1_ref[...], preferred_element_type=jnp.float32)
    h = jnp.maximum(h + b1_ref[...], 0.0).astype(jnp.bfloat16)
    y = jnp.dot(h, w2_ref[...], preferred_element_type=jnp.float32)
    o_ref[...] = y + b2_ref[...]


def _fc(xf, w1b, b1, w2b, b2):
    bp = xf.shape[0]
    return pl.pallas_call(
        _fc_k,
        out_shape=jax.ShapeDtypeStruct((bp, 10), jnp.float32),
        grid_spec=pltpu.PrefetchScalarGridSpec(
            num_scalar_prefetch=0,
            grid=(bp // TBF,),
            in_specs=[
                pl.BlockSpec((TBF, FC_IN), lambda b: (b, 0)),
                pl.BlockSpec((FC_IN, 128), lambda b: (0, 0)),
                pl.BlockSpec((1, 128), lambda b: (0, 0)),
                pl.BlockSpec((128, 10), lambda b: (0, 0)),
                pl.BlockSpec((1, 10), lambda b: (0, 0)),
            ],
            out_specs=pl.BlockSpec((TBF, 10), lambda b: (b, 0)),
        ),
        compiler_params=pltpu.CompilerParams(
            dimension_semantics=("parallel",)),
    )(xf, w1b, b1, w2b, b2)


def kernel(w1, b1, w2, b2, fc1_w, fc1_b, fc2_w, fc2_b, mask_l, mask_r, x):
    B = x.shape[0]
    xf = x.reshape(B, NPIX, 1)
    bp = ((B + TB - 1) // TB) * TB
    if bp != B:
        xf = jnp.pad(xf, ((0, bp - B), (0, 0), (0, 0)))
    tml = jnp.tile(jnp.pad(mask_l, ((0, MARGIN), (0, 0))),
                   (TB, 1)).astype(jnp.bfloat16)
    tmr = jnp.tile(jnp.pad(mask_r, ((0, MARGIN), (0, 0))),
                   (TB, 1)).astype(jnp.bfloat16)
    w2b_ = w2.astype(jnp.bfloat16)
    w2p = jnp.concatenate([w2b_[0:8:2], w2b_[1:8:2]], axis=1)
    w2l = w2b_[8]
    sidx = jnp.arange(NPOOL)
    qs = 4 * W * (sidx // WO) + 4 * (sidx % WO)
    pm = (jnp.arange(PCOLS)[None, :] == qs[:, None]).astype(jnp.bfloat16)
    pm = jnp.pad(pm, ((0, PROWS - NPOOL), (0, 0)))
    pooled = _conv_pool(xf, tml, tmr, w1, b1, w2p, w2l, b2, pm)[:B]

    flat = pooled.reshape(B, FC_IN)
    bf = ((B + TBF - 1) // TBF) * TBF
    if bf != B:
        flat = jnp.pad(flat, ((0, bf - B), (0, 0)))
    logits = _fc(flat, fc1_w.astype(jnp.bfloat16), fc1_b,
                 fc2_w.astype(jnp.bfloat16), fc2_b)[:B]
    return logits


# speedup vs baseline: 4.9271x; 4.8513x over previous
import jax
import jax.numpy as jnp
from jax.experimental import pallas as pl
from jax.experimental.pallas import tpu as pltpu

H = W = 28
C1, C2 = 32, 64
L1 = W * C1
L2 = W * C2
HO = WO = 7
FC_IN = 3136
GAP = 4
SR = H + GAP
TB = 8
R = TB * SR
XROWS = R + 8
POUT = 448
TBF = 256


def _conv_pool_k(x_ref, b1c_ref, w2c_ref, b1t_ref, b2t_ref, pv_ref, ps_ref,
                 o_ref, xs_ref, h1_ref, h2_ref):
    xs_ref[...] = jnp.zeros_like(xs_ref)
    h1_ref[...] = jnp.zeros_like(h1_ref)
    h2_ref[...] = jnp.zeros_like(h2_ref)
    for i in range(TB):
        xs_ref[pl.ds(4 + i * SR, H), pl.ds(0, W)] = x_ref[i].astype(jnp.bfloat16)

    xcat = jnp.concatenate(
        [xs_ref[pl.ds(3 + k, R), :] for k in range(3)], axis=1)
    a1 = jnp.dot(xcat, b1c_ref[...], preferred_element_type=jnp.float32)
    h1 = jnp.maximum(a1 + b1t_ref[...], 0.0).astype(jnp.bfloat16)
    for i in range(TB):
        h1_ref[pl.ds(4 + i * SR, H), :] = h1[i * SR:i * SR + H, :]

    hcat = jnp.concatenate(
        [h1_ref[pl.ds(3 + k, R), :] for k in range(3)], axis=1)
    a2 = jnp.dot(hcat, w2c_ref[...], preferred_element_type=jnp.float32)
    h2 = jnp.maximum(a2 + b2t_ref[...], 0.0).astype(jnp.bfloat16)
    h2_ref[pl.ds(4, R), :] = h2

    vm = jnp.maximum(
        jnp.maximum(h2_ref[pl.ds(4, R), :], h2_ref[pl.ds(5, R), :]),
        jnp.maximum(h2_ref[pl.ds(6, R), :], h2_ref[pl.ds(7, R), :]))
    tmp = jnp.dot(pv_ref[...], vm, preferred_element_type=jnp.float32)
    tpad = jnp.concatenate(
        [tmp, jnp.zeros((TB * 8, 3 * C2), jnp.float32)], axis=1)
    hv = jnp.maximum(
        jnp.maximum(tpad[:, 0 * C2:0 * C2 + L2], tpad[:, 1 * C2:1 * C2 + L2]),
        jnp.maximum(tpad[:, 2 * C2:2 * C2 + L2], tpad[:, 3 * C2:3 * C2 + L2]))
    res = jnp.dot(hv.astype(jnp.bfloat16), ps_ref[...],
                  preferred_element_type=jnp.float32)
    for i in range(TB):
        o_ref[i] = res[i * 8:i * 8 + HO, :POUT].astype(jnp.bfloat16)


def _conv_pool(xp, b1c, w2c, b1t, b2t, pv, ps):
    bp = xp.shape[0]
    return pl.pallas_call(
        _conv_pool_k,
        out_shape=jax.ShapeDtypeStruct((bp, HO, POUT), jnp.bfloat16),
        grid_spec=pltpu.PrefetchScalarGridSpec(
            num_scalar_prefetch=0,
            grid=(bp // TB,),
            in_specs=[
                pl.BlockSpec((TB, H, W), lambda b: (b, 0, 0)),
                pl.BlockSpec((96, L1), lambda b: (0, 0)),
                pl.BlockSpec((3 * L1, L2), lambda b: (0, 0)),
                pl.BlockSpec((1, L1), lambda b: (0, 0)),
                pl.BlockSpec((1, L2), lambda b: (0, 0)),
                pl.BlockSpec((TB * 8, R), lambda b: (0, 0)),
                pl.BlockSpec((L2, 512), lambda b: (0, 0)),
            ],
            out_specs=pl.BlockSpec((TB, HO, POUT), lambda b: (b, 0, 0)),
            scratch_shapes=[
                pltpu.VMEM((XROWS, C1), jnp.bfloat16),
                pltpu.VMEM((XROWS, L1), jnp.bfloat16),
                pltpu.VMEM((XROWS, L2), jnp.bfloat16),
            ]),
        compiler_params=pltpu.CompilerParams(
            dimension_semantics=("parallel",)),
    )(xp, b1c, w2c, b1t, b2t, pv, ps)


def _fc_k(x_ref, w1_ref, b1_ref, w2_ref, b2_ref, o_ref):
    h = jnp.dot(x_ref[...], w1_ref[...], preferred_element_type=jnp.float32)
    h = jnp.maximum(h + b1_ref[...], 0.0).astype(jnp.bfloat16)
    y = jnp.dot(h, w2_ref[...], preferred_element_type=jnp.float32)
    o_ref[...] = y + b2_ref[...]


def _fc(xf, w1b, b1, w2b, b2):
    bp = xf.shape[0]
    return pl.pallas_call(
        _fc_k,
        out_shape=jax.ShapeDtypeStruct((bp, 10), jnp.float32),
        grid_spec=pltpu.PrefetchScalarGridSpec(
            num_scalar_prefetch=0,
            grid=(bp // TBF,),
            in_specs=[
                pl.BlockSpec((TBF, FC_IN), lambda b: (b, 0)),
                pl.BlockSpec((FC_IN, 128), lambda b: (0, 0)),
                pl.BlockSpec((1, 128), lambda b: (0, 0)),
                pl.BlockSpec((128, 10), lambda b: (0, 0)),
                pl.BlockSpec((1, 10), lambda b: (0, 0)),
            ],
            out_specs=pl.BlockSpec((TBF, 10), lambda b: (b, 0)),
        ),
        compiler_params=pltpu.CompilerParams(
            dimension_semantics=("parallel",)),
    )(xf, w1b, b1, w2b, b2)


def kernel(w1, b1, w2, b2, fc1_w, fc1_b, fc2_w, fc2_b, mask_l, mask_r, x):
    B = x.shape[0]
    xi = x.reshape(B, H, W)
    bp = ((B + TB - 1) // TB) * TB
    if bp != B:
        xi = jnp.pad(xi, ((0, bp - B), (0, 0), (0, 0)))

    eyes = [jnp.eye(W, k=1 - dj, dtype=jnp.float32) for dj in range(3)]
    b1rows = []
    for k in range(3):
        bd = sum(jnp.einsum('pj,c->pjc', eyes[dj], w1[k * 3 + dj, :C1])
                 for dj in range(3))
        b1rows.append(jnp.pad(bd.reshape(W, L1), ((0, C1 - W), (0, 0))))
    b1c = jnp.concatenate(b1rows, axis=0).astype(jnp.bfloat16)

    w2rows = []
    for k in range(3):
        wd = sum(jnp.einsum('pj,co->pcjo', eyes[dj], w2[k * 3 + dj, :C1, :C2])
                 for dj in range(3))
        w2rows.append(wd.reshape(L1, L2))
    w2c = jnp.concatenate(w2rows, axis=0).astype(jnp.bfloat16)

    b1t = jnp.tile(b1[:, :C1], (1, W))
    b2t = jnp.tile(b2[:, :C2], (1, W))

    ri = jnp.arange(TB * 8)
    pv = jax.nn.one_hot((ri // 8) * SR + 4 * (ri % 8), R,
                        dtype=jnp.bfloat16)
    li = jnp.arange(512)
    ps = (jnp.arange(L2)[:, None] ==
          (256 * (li // C2) + li % C2)[None, :]).astype(jnp.bfloat16)

    pooled = _conv_pool(xi, b1c, w2c, b1t, b2t, pv, ps)[:B]
    flat = pooled.reshape(B, FC_IN)
    bf = ((B + TBF - 1) // TBF) * TBF
    if bf != B:
        flat = jnp.pad(flat, ((0, bf - B), (0, 0)))
    logits = _fc(flat, fc1_w.astype(jnp.bfloat16), fc1_b,
                 fc2_w.astype(jnp.bfloat16), fc2_b)[:B]
    return logits


# speedup vs baseline: 7.0749x; 1.4359x over previous
import jax
import jax.numpy as jnp
from jax.experimental import pallas as pl
from jax.experimental.pallas import tpu as pltpu

H = W = 28
C1, C2 = 32, 64
L1 = W * C1
L2 = W * C2
HO = WO = 7
POUT = WO * C2
TB = 16
R = H * TB
TBF = 256


def _conv_pool_k(x_ref, b1c_ref, w2c_ref, b1t_ref, b2t_ref, ps_ref,
                 o_ref, xs_ref, h1_ref):
    xv = x_ref[...].reshape(R, W).astype(jnp.bfloat16)
    xs_ref[pl.ds(TB, R), :] = jnp.concatenate(
        [xv, jnp.zeros((R, C1 - W), jnp.bfloat16)], axis=1)
    zx = jnp.zeros((TB, C1), jnp.bfloat16)
    xs_ref[pl.ds(0, TB), :] = zx
    xs_ref[pl.ds((H + 1) * TB, TB), :] = zx

    xcat = jnp.concatenate(
        [xs_ref[pl.ds(k * TB, R), :] for k in range(3)], axis=1)
    a1 = jnp.dot(xcat, b1c_ref[...], preferred_element_type=jnp.float32)
    h1 = jnp.maximum(a1 + b1t_ref[...], 0.0).astype(jnp.bfloat16)
    h1_ref[pl.ds(TB, R), :] = h1
    zh = jnp.zeros((TB, L1), jnp.bfloat16)
    h1_ref[pl.ds(0, TB), :] = zh
    h1_ref[pl.ds((H + 1) * TB, TB), :] = zh

    hcat = jnp.concatenate(
        [h1_ref[pl.ds(k * TB, R), :] for k in range(3)], axis=1)
    a2 = jnp.dot(hcat, w2c_ref[...], preferred_element_type=jnp.float32)
    h2 = jnp.maximum(a2 + b2t_ref[...], 0.0).astype(jnp.bfloat16)

    pr = (4 * (HO - 1) + 1) * TB
    vm = jnp.maximum(
        jnp.maximum(h2[0:pr], h2[TB:TB + pr]),
        jnp.maximum(h2[2 * TB:2 * TB + pr], h2[3 * TB:3 * TB + pr]))
    vsel = jnp.concatenate(
        [vm[4 * oy * TB:(4 * oy + 1) * TB] for oy in range(HO)], axis=0)
    tpad = jnp.concatenate(
        [vsel, jnp.zeros((HO * TB, 3 * C2), jnp.bfloat16)], axis=1)
    hv = jnp.maximum(
        jnp.maximum(tpad[:, 0:L2], tpad[:, C2:C2 + L2]),
        jnp.maximum(tpad[:, 2 * C2:2 * C2 + L2], tpad[:, 3 * C2:3 * C2 + L2]))
    res = jnp.dot(hv, ps_ref[...], preferred_element_type=jnp.float32)
    for oy in range(HO):
        o_ref[oy] = res[oy * TB:(oy + 1) * TB, :POUT].astype(jnp.bfloat16)


def _conv_pool(xt, b1c, w2c, b1t, b2t, ps):
    bp = xt.shape[1]
    return pl.pallas_call(
        _conv_pool_k,
        out_shape=jax.ShapeDtypeStruct((HO, bp, POUT), jnp.bfloat16),
        grid_spec=pltpu.PrefetchScalarGridSpec(
            num_scalar_prefetch=0,
            grid=(bp // TB,),
            in_specs=[
                pl.BlockSpec((H, TB, W), lambda b: (0, b, 0)),
                pl.BlockSpec((96, L1), lambda b: (0, 0)),
                pl.BlockSpec((3 * L1, L2), lambda b: (0, 0)),
                pl.BlockSpec((1, L1), lambda b: (0, 0)),
                pl.BlockSpec((1, L2), lambda b: (0, 0)),
                pl.BlockSpec((L2, 512), lambda b: (0, 0)),
            ],
            out_specs=pl.BlockSpec((HO, TB, POUT), lambda b: (0, b, 0)),
            scratch_shapes=[
                pltpu.VMEM(((H + 2) * TB, C1), jnp.bfloat16),
                pltpu.VMEM(((H + 2) * TB, L1), jnp.bfloat16),
            ]),
        compiler_params=pltpu.CompilerParams(
            dimension_semantics=("parallel",)),
    )(xt, b1c, w2c, b1t, b2t, ps)


def _fc_k(x_ref, w1_ref, b1_ref, w2_ref, b2_ref, o_ref):
    h = jnp.dot(x_ref[0], w1_ref[0], preferred_element_type=jnp.float32)
    for oy in range(1, HO):
        h = h + jnp.dot(x_ref[oy], w1_ref[oy],
                        preferred_element_type=jnp.float32)
    h = jnp.maximum(h + b1_ref[...], 0.0).astype(jnp.bfloat16)
    y = jnp.dot(h, w2_ref[...], preferred_element_type=jnp.float32)
    o_ref[...] = y + b2_ref[...]


def _fc(xp, w1b, b1, w2b, b2):
    bp = xp.shape[1]
    return pl.pallas_call(
        _fc_k,
        out_shape=jax.ShapeDtypeStruct((bp, 10), jnp.float32),
        grid_spec=pltpu.PrefetchScalarGridSpec(
            num_scalar_prefetch=0,
            grid=(bp // TBF,),
            in_specs=[
                pl.BlockSpec((HO, TBF, POUT), lambda b: (0, b, 0)),
                pl.BlockSpec((HO, POUT, 128), lambda b: (0, 0, 0)),
                pl.BlockSpec((1, 128), lambda b: (0, 0)),
                pl.BlockSpec((128, 10), lambda b: (0, 0)),
                pl.BlockSpec((1, 10), lambda b: (0, 0)),
            ],
            out_specs=pl.BlockSpec((TBF, 10), lambda b: (b, 0)),
        ),
        compiler_params=pltpu.CompilerParams(
            dimension_semantics=("parallel",)),
    )(xp, w1b, b1, w2b, b2)


def kernel(w1, b1, w2, b2, fc1_w, fc1_b, fc2_w, fc2_b, mask_l, mask_r, x):
    B = x.shape[0]
    bp = ((B + TBF - 1) // TBF) * TBF
    xi = x.reshape(B, H, W)
    if bp != B:
        xi = jnp.pad(xi, ((0, bp - B), (0, 0), (0, 0)))
    xt = jnp.transpose(xi, (1, 0, 2))

    eyes = [jnp.eye(W, k=1 - dj, dtype=jnp.float32) for dj in range(3)]
    b1rows = []
    for k in range(3):
        bd = sum(jnp.einsum('pj,c->pjc', eyes[dj], w1[k * 3 + dj, :C1])
                 for dj in range(3))
        b1rows.append(jnp.pad(bd.reshape(W, L1), ((0, C1 - W), (0, 0))))
    b1c = jnp.concatenate(b1rows, axis=0).astype(jnp.bfloat16)

    w2rows = []
    for k in range(3):
        wd = sum(jnp.einsum('pj,co->pcjo', eyes[dj], w2[k * 3 + dj, :C1, :C2])
                 for dj in range(3))
        w2rows.append(wd.reshape(L1, L2))
    w2c = jnp.concatenate(w2rows, axis=0).astype(jnp.bfloat16)

    b1t = jnp.tile(b1[:, :C1], (1, W))
    b2t = jnp.tile(b2[:, :C2], (1, W))

    li = jnp.arange(512)
    ps = (jnp.arange(L2)[:, None] ==
          (4 * C2 * (li // C2) + li % C2)[None, :]).astype(jnp.bfloat16)

    pooled = _conv_pool(xt, b1c, w2c, b1t, b2t, ps)
    logits = _fc(pooled, fc1_w.reshape(HO, POUT, 128).astype(jnp.bfloat16),
                 fc1_b, fc2_w.astype(jnp.bfloat16), fc2_b)[:B]
    return logits


# speedup vs baseline: 7.4564x; 1.0539x over previous
import jax
import jax.numpy as jnp
from jax.experimental import pallas as pl
from jax.experimental.pallas import tpu as pltpu

H = W = 28
C1, C2 = 32, 64
L1 = W * C1
L2 = W * C2
HO = WO = 7
POUT = WO * C2
TB = 32
R = H * TB
TBF = 256


def _conv_pool_k(x_ref, b1c_ref, w2c_ref, b1t_ref, b2t_ref, ps_ref,
                 o_ref, xs_ref, h1_ref):
    xv = x_ref[...].reshape(R, W).astype(jnp.bfloat16)
    xs_ref[pl.ds(TB, R), :] = jnp.concatenate(
        [xv, jnp.zeros((R, C1 - W), jnp.bfloat16)], axis=1)
    zx = jnp.zeros((TB, C1), jnp.bfloat16)
    xs_ref[pl.ds(0, TB), :] = zx
    xs_ref[pl.ds((H + 1) * TB, TB), :] = zx

    xcat = jnp.concatenate(
        [xs_ref[pl.ds(k * TB, R), :] for k in range(3)], axis=1)
    a1 = jnp.dot(xcat, b1c_ref[...], preferred_element_type=jnp.float32)
    h1 = jnp.maximum(a1 + b1t_ref[...], 0.0).astype(jnp.bfloat16)
    h1_ref[pl.ds(TB, R), :] = h1
    zh = jnp.zeros((TB, L1), jnp.bfloat16)
    h1_ref[pl.ds(0, TB), :] = zh
    h1_ref[pl.ds((H + 1) * TB, TB), :] = zh

    hcat = jnp.concatenate(
        [h1_ref[pl.ds(k * TB, R), :] for k in range(3)], axis=1)
    a2 = jnp.dot(hcat, w2c_ref[...], preferred_element_type=jnp.float32)
    h2 = jnp.maximum(a2 + b2t_ref[...], 0.0).astype(jnp.bfloat16)

    pr = (4 * (HO - 1) + 1) * TB
    vm = jnp.maximum(
        jnp.maximum(h2[0:pr], h2[TB:TB + pr]),
        jnp.maximum(h2[2 * TB:2 * TB + pr], h2[3 * TB:3 * TB + pr]))
    vsel = jnp.concatenate(
        [vm[4 * oy * TB:(4 * oy + 1) * TB] for oy in range(HO)], axis=0)
    tpad = jnp.concatenate(
        [vsel, jnp.zeros((HO * TB, 3 * C2), jnp.bfloat16)], axis=1)
    hv = jnp.maximum(
        jnp.maximum(tpad[:, 0:L2], tpad[:, C2:C2 + L2]),
        jnp.maximum(tpad[:, 2 * C2:2 * C2 + L2], tpad[:, 3 * C2:3 * C2 + L2]))
    res = jnp.dot(hv, ps_ref[...], preferred_element_type=jnp.float32)
    for oy in range(HO):
        o_ref[oy] = res[oy * TB:(oy + 1) * TB, :POUT].astype(jnp.bfloat16)


def _conv_pool(xt, b1c, w2c, b1t, b2t, ps):
    bp = xt.shape[1]
    return pl.pallas_call(
        _conv_pool_k,
        out_shape=jax.ShapeDtypeStruct((HO, bp, POUT), jnp.bfloat16),
        grid_spec=pltpu.PrefetchScalarGridSpec(
            num_scalar_prefetch=0,
            grid=(bp // TB,),
            in_specs=[
                pl.BlockSpec((H, TB, W), lambda b: (0, b, 0)),
                pl.BlockSpec((96, L1), lambda b: (0, 0)),
                pl.BlockSpec((3 * L1, L2), lambda b: (0, 0)),
                pl.BlockSpec((1, L1), lambda b: (0, 0)),
                pl.BlockSpec((1, L2), lambda b: (0, 0)),
                pl.BlockSpec((L2, 512), lambda b: (0, 0)),
            ],
            out_specs=pl.BlockSpec((HO, TB, POUT), lambda b: (0, b, 0)),
            scratch_shapes=[
                pltpu.VMEM(((H + 2) * TB, C1), jnp.bfloat16),
                pltpu.VMEM(((H + 2) * TB, L1), jnp.bfloat16),
            ]),
        compiler_params=pltpu.CompilerParams(
            dimension_semantics=("parallel",)),
    )(xt, b1c, w2c, b1t, b2t, ps)


def _fc_k(x_ref, w1_ref, b1_ref, w2_ref, b2_ref, o_ref):
    h = jnp.dot(x_ref[0], w1_ref[0], preferred_element_type=jnp.float32)
    for oy in range(1, HO):
        h = h + jnp.dot(x_ref[oy], w1_ref[oy],
                        preferred_element_type=jnp.float32)
    h = jnp.maximum(h + b1_ref[...], 0.0).astype(jnp.bfloat16)
    y = jnp.dot(h, w2_ref[...], preferred_element_type=jnp.float32)
    o_ref[...] = y + b2_ref[...]


def _fc(xp, w1b, b1, w2b, b2):
    bp = xp.shape[1]
    return pl.pallas_call(
        _fc_k,
        out_shape=jax.ShapeDtypeStruct((bp, 10), jnp.float32),
        grid_spec=pltpu.PrefetchScalarGridSpec(
            num_scalar_prefetch=0,
            grid=(bp // TBF,),
            in_specs=[
                pl.BlockSpec((HO, TBF, POUT), lambda b: (0, b, 0)),
                pl.BlockSpec((HO, POUT, 128), lambda b: (0, 0, 0)),
                pl.BlockSpec((1, 128), lambda b: (0, 0)),
                pl.BlockSpec((128, 10), lambda b: (0, 0)),
                pl.BlockSpec((1, 10), lambda b: (0, 0)),
            ],
            out_specs=pl.BlockSpec((TBF, 10), lambda b: (b, 0)),
        ),
        compiler_params=pltpu.CompilerParams(
            dimension_semantics=("parallel",)),
    )(xp, w1b, b1, w2b, b2)


def kernel(w1, b1, w2, b2, fc1_w, fc1_b, fc2_w, fc2_b, mask_l, mask_r, x):
    B = x.shape[0]
    bp = ((B + TBF - 1) // TBF) * TBF
    xi = x.reshape(B, H, W)
    if bp != B:
        xi = jnp.pad(xi, ((0, bp - B), (0, 0), (0, 0)))
    xt = jnp.transpose(xi, (1, 0, 2))

    eyes = [jnp.eye(W, k=1 - dj, dtype=jnp.float32) for dj in range(3)]
    b1rows = []
    for k in range(3):
        bd = sum(jnp.einsum('pj,c->pjc', eyes[dj], w1[k * 3 + dj, :C1])
                 for dj in range(3))
        b1rows.append(jnp.pad(bd.reshape(W, L1), ((0, C1 - W), (0, 0))))
    b1c = jnp.concatenate(b1rows, axis=0).astype(jnp.bfloat16)

    w2rows = []
    for k in range(3):
        wd = sum(jnp.einsum('pj,co->pcjo', eyes[dj], w2[k * 3 + dj, :C1, :C2])
                 for dj in range(3))
        w2rows.append(wd.reshape(L1, L2))
    w2c = jnp.concatenate(w2rows, axis=0).astype(jnp.bfloat16)

    b1t = jnp.tile(b1[:, :C1], (1, W))
    b2t = jnp.tile(b2[:, :C2], (1, W))

    li = jnp.arange(512)
    ps = (jnp.arange(L2)[:, None] ==
          (4 * C2 * (li // C2) + li % C2)[None, :]).astype(jnp.bfloat16)

    pooled = _conv_pool(xt, b1c, w2c, b1t, b2t, ps)
    logits = _fc(pooled, fc1_w.reshape(HO, POUT, 128).astype(jnp.bfloat16),
                 fc1_b, fc2_w.astype(jnp.bfloat16), fc2_b)[:B]
    return logits


# speedup vs baseline: 7.5899x; 1.0179x over previous
import jax
import jax.numpy as jnp
from jax.experimental import pallas as pl
from jax.experimental.pallas import tpu as pltpu

H = W = 28
C1, C2 = 32, 64
L1 = W * C1
L2 = W * C2
HO = WO = 7
POUT = WO * C2
TB = 64
R = H * TB
TBF = 256


def _conv_pool_k(x_ref, b1c_ref, w2c_ref, b1t_ref, b2t_ref, ps_ref,
                 o_ref, xs_ref, h1_ref):
    xv = x_ref[...].reshape(R, W).astype(jnp.bfloat16)
    xs_ref[pl.ds(TB, R), :] = jnp.concatenate(
        [xv, jnp.zeros((R, C1 - W), jnp.bfloat16)], axis=1)
    zx = jnp.zeros((TB, C1), jnp.bfloat16)
    xs_ref[pl.ds(0, TB), :] = zx
    xs_ref[pl.ds((H + 1) * TB, TB), :] = zx

    xcat = jnp.concatenate(
        [xs_ref[pl.ds(k * TB, R), :] for k in range(3)], axis=1)
    a1 = jnp.dot(xcat, b1c_ref[...], preferred_element_type=jnp.float32)
    h1 = jnp.maximum(a1 + b1t_ref[...], 0.0).astype(jnp.bfloat16)
    h1_ref[pl.ds(TB, R), :] = h1
    zh = jnp.zeros((TB, L1), jnp.bfloat16)
    h1_ref[pl.ds(0, TB), :] = zh
    h1_ref[pl.ds((H + 1) * TB, TB), :] = zh

    hcat = jnp.concatenate(
        [h1_ref[pl.ds(k * TB, R), :] for k in range(3)], axis=1)
    a2 = jnp.dot(hcat, w2c_ref[...], preferred_element_type=jnp.float32)
    h2 = jnp.maximum(a2 + b2t_ref[...], 0.0).astype(jnp.bfloat16)

    pr = (4 * (HO - 1) + 1) * TB
    vm = jnp.maximum(
        jnp.maximum(h2[0:pr], h2[TB:TB + pr]),
        jnp.maximum(h2[2 * TB:2 * TB + pr], h2[3 * TB:3 * TB + pr]))
    vsel = jnp.concatenate(
        [vm[4 * oy * TB:(4 * oy + 1) * TB] for oy in range(HO)], axis=0)
    tpad = jnp.concatenate(
        [vsel, jnp.zeros((HO * TB, 3 * C2), jnp.bfloat16)], axis=1)
    hv = jnp.maximum(
        jnp.maximum(tpad[:, 0:L2], tpad[:, C2:C2 + L2]),
        jnp.maximum(tpad[:, 2 * C2:2 * C2 + L2], tpad[:, 3 * C2:3 * C2 + L2]))
    res = jnp.dot(hv, ps_ref[...], preferred_element_type=jnp.float32)
    for oy in range(HO):
        o_ref[oy] = res[oy * TB:(oy + 1) * TB, :POUT].astype(jnp.bfloat16)


def _conv_pool(xt, b1c, w2c, b1t, b2t, ps):
    bp = xt.shape[1]
    return pl.pallas_call(
        _conv_pool_k,
        out_shape=jax.ShapeDtypeStruct((HO, bp, POUT), jnp.bfloat16),
        grid_spec=pltpu.PrefetchScalarGridSpec(
            num_scalar_prefetch=0,
            grid=(bp // TB,),
            in_specs=[
                pl.BlockSpec((H, TB, W), lambda b: (0, b, 0)),
                pl.BlockSpec((96, L1), lambda b: (0, 0)),
                pl.BlockSpec((3 * L1, L2), lambda b: (0, 0)),
                pl.BlockSpec((1, L1), lambda b: (0, 0)),
                pl.BlockSpec((1, L2), lambda b: (0, 0)),
                pl.BlockSpec((L2, 512), lambda b: (0, 0)),
            ],
            out_specs=pl.BlockSpec((HO, TB, POUT), lambda b: (0, b, 0)),
            scratch_shapes=[
                pltpu.VMEM(((H + 2) * TB, C1), jnp.bfloat16),
                pltpu.VMEM(((H + 2) * TB, L1), jnp.bfloat16),
            ]),
        compiler_params=pltpu.CompilerParams(
            dimension_semantics=("parallel",)),
    )(xt, b1c, w2c, b1t, b2t, ps)


def _fc_k(x_ref, w1_ref, b1_ref, w2_ref, b2_ref, o_ref):
    h = jnp.dot(x_ref[0], w1_ref[0], preferred_element_type=jnp.float32)
    for oy in range(1, HO):
        h = h + jnp.dot(x_ref[oy], w1_ref[oy],
                        preferred_element_type=jnp.float32)
    h = jnp.maximum(h + b1_ref[...], 0.0).astype(jnp.bfloat16)
    y = jnp.dot(h, w2_ref[...], preferred_element_type=jnp.float32)
    o_ref[...] = y + b2_ref[...]


def _fc(xp, w1b, b1, w2b, b2):
    bp = xp.shape[1]
    return pl.pallas_call(
        _fc_k,
        out_shape=jax.ShapeDtypeStruct((bp, 10), jnp.float32),
        grid_spec=pltpu.PrefetchScalarGridSpec(
            num_scalar_prefetch=0,
            grid=(bp // TBF,),
            in_specs=[
                pl.BlockSpec((HO, TBF, POUT), lambda b: (0, b, 0)),
                pl.BlockSpec((HO, POUT, 128), lambda b: (0, 0, 0)),
                pl.BlockSpec((1, 128), lambda b: (0, 0)),
                pl.BlockSpec((128, 10), lambda b: (0, 0)),
                pl.BlockSpec((1, 10), lambda b: (0, 0)),
            ],
            out_specs=pl.BlockSpec((TBF, 10), lambda b: (b, 0)),
        ),
        compiler_params=pltpu.CompilerParams(
            dimension_semantics=("parallel",)),
    )(xp, w1b, b1, w2b, b2)


def kernel(w1, b1, w2, b2, fc1_w, fc1_b, fc2_w, fc2_b, mask_l, mask_r, x):
    B = x.shape[0]
    bp = ((B + TBF - 1) // TBF) * TBF
    xi = x.reshape(B, H, W)
    if bp != B:
        xi = jnp.pad(xi, ((0, bp - B), (0, 0), (0, 0)))
    xt = jnp.transpose(xi, (1, 0, 2))

    eyes = [jnp.eye(W, k=1 - dj, dtype=jnp.float32) for dj in range(3)]
    b1rows = []
    for k in range(3):
        bd = sum(jnp.einsum('pj,c->pjc', eyes[dj], w1[k * 3 + dj, :C1])
                 for dj in range(3))
        b1rows.append(jnp.pad(bd.reshape(W, L1), ((0, C1 - W), (0, 0))))
    b1c = jnp.concatenate(b1rows, axis=0).astype(jnp.bfloat16)

    w2rows = []
    for k in range(3):
        wd = sum(jnp.einsum('pj,co->pcjo', eyes[dj], w2[k * 3 + dj, :C1, :C2])
                 for dj in range(3))
        w2rows.append(wd.reshape(L1, L2))
    w2c = jnp.concatenate(w2rows, axis=0).astype(jnp.bfloat16)

    b1t = jnp.tile(b1[:, :C1], (1, W))
    b2t = jnp.tile(b2[:, :C2], (1, W))

    li = jnp.arange(512)
    ps = (jnp.arange(L2)[:, None] ==
          (4 * C2 * (li // C2) + li % C2)[None, :]).astype(jnp.bfloat16)

    pooled = _conv_pool(xt, b1c, w2c, b1t, b2t, ps)
    logits = _fc(pooled, fc1_w.reshape(HO, POUT, 128).astype(jnp.bfloat16),
                 fc1_b, fc2_w.astype(jnp.bfloat16), fc2_b)[:B]
    return logits
